# Initial kernel scaffold; baseline (speedup 1.0000x reference)
import jax
import jax.numpy as jnp
from jax.experimental import pallas as pl


def kernel(x, t, emb_weight, W, b):
    raise NotImplementedError("write your pallas kernel here")


# speedup vs baseline: 8.0927x; 8.0927x over previous
import functools

import jax
import jax.numpy as jnp
from jax import lax
from jax.experimental import pallas as pl
from jax.experimental.pallas import tpu as pltpu

N_HID = 128
MAX_LEN = 100
BLK = 2048


def _proj_table_kernel(emb_ref, w_ref, b_ref, p_ref):
    p_ref[...] = lax.dot_general(
        emb_ref[...], w_ref[...],
        dimension_numbers=(((1,), (1,)), ((), ())),
        preferred_element_type=jnp.float32,
    ) + b_ref[...]


def _gather_add_kernel(idx_ref, x_ref, p_ref, out_ref):
    idx = idx_ref[0, 0, :] * 2
    rows = lax.broadcasted_iota(jnp.int32, (MAX_LEN, BLK), 0)
    onehot = (rows == idx[None, :]).astype(jnp.float32)
    proj = lax.dot_general(
        p_ref[...], onehot,
        dimension_numbers=(((0,), (0,)), ((), ())),
        preferred_element_type=jnp.float32,
    )
    out_ref[...] = x_ref[...] + proj


def kernel(x, t, emb_weight, W, b):
    n = x.shape[1]
    num_blk = n // BLK

    p = pl.pallas_call(
        _proj_table_kernel,
        out_shape=jax.ShapeDtypeStruct((MAX_LEN, N_HID), jnp.float32),
    )(emb_weight, W, b.reshape(1, N_HID))

    t3 = t.reshape(num_blk, 1, BLK)

    out = pl.pallas_call(
        _gather_add_kernel,
        grid=(num_blk,),
        in_specs=[
            pl.BlockSpec((1, 1, BLK), lambda i: (i, 0, 0)),
            pl.BlockSpec((N_HID, BLK), lambda i: (0, i)),
            pl.BlockSpec((MAX_LEN, N_HID), lambda i: (0, 0)),
        ],
        out_specs=pl.BlockSpec((N_HID, BLK), lambda i: (0, i)),
        out_shape=jax.ShapeDtypeStruct((N_HID, n), jnp.float32),
        compiler_params=pltpu.CompilerParams(
            dimension_semantics=("arbitrary",),
        ),
    )(t3, x, p)
    return out


# speedup vs baseline: 11.3938x; 1.4079x over previous
import functools

import jax
import jax.numpy as jnp
from jax import lax
from jax.experimental import pallas as pl
from jax.experimental.pallas import tpu as pltpu

N_HID = 128
MAX_LEN = 100
BLK = 4096


def _proj_table_kernel(emb_ref, w_ref, b_ref, p_ref):
    p_ref[...] = lax.dot_general(
        emb_ref[...], w_ref[...],
        dimension_numbers=(((1,), (1,)), ((), ())),
        preferred_element_type=jnp.float32,
    ) + b_ref[...]


def _gather_add_kernel(idx_ref, x_ref, p_ref, out_ref):
    idx = idx_ref[0, 0, :] * 2
    rows = lax.broadcasted_iota(jnp.int32, (MAX_LEN, BLK), 0)
    onehot = (rows == idx[None, :]).astype(jnp.float32)
    proj = lax.dot_general(
        p_ref[...], onehot,
        dimension_numbers=(((0,), (0,)), ((), ())),
        preferred_element_type=jnp.float32,
    )
    out_ref[...] = x_ref[...] + proj


def kernel(x, t, emb_weight, W, b):
    n = x.shape[1]
    num_blk = n // BLK

    p = pl.pallas_call(
        _proj_table_kernel,
        out_shape=jax.ShapeDtypeStruct((MAX_LEN, N_HID), jnp.float32),
    )(emb_weight, W, b.reshape(1, N_HID))

    t3 = t.reshape(num_blk, 1, BLK)

    out = pl.pallas_call(
        _gather_add_kernel,
        grid=(num_blk,),
        in_specs=[
            pl.BlockSpec((1, 1, BLK), lambda i: (i, 0, 0)),
            pl.BlockSpec((N_HID, BLK), lambda i: (0, i)),
            pl.BlockSpec((MAX_LEN, N_HID), lambda i: (0, 0)),
        ],
        out_specs=pl.BlockSpec((N_HID, BLK), lambda i: (0, i)),
        out_shape=jax.ShapeDtypeStruct((N_HID, n), jnp.float32),
        compiler_params=pltpu.CompilerParams(
            dimension_semantics=("arbitrary",),
        ),
    )(t3, x, p)
    return out


# speedup vs baseline: 12.8915x; 1.1314x over previous
import functools

import jax
import jax.numpy as jnp
from jax import lax
from jax.experimental import pallas as pl
from jax.experimental.pallas import tpu as pltpu

N_HID = 128
MAX_LEN = 100
BLK = 8192


def _proj_table_kernel(emb_ref, w_ref, b_ref, p_ref):
    p_ref[...] = lax.dot_general(
        emb_ref[...], w_ref[...],
        dimension_numbers=(((1,), (1,)), ((), ())),
        preferred_element_type=jnp.float32,
    ) + b_ref[...]


def _gather_add_kernel(idx_ref, x_ref, p_ref, out_ref):
    idx = idx_ref[0, 0, :] * 2
    rows = lax.broadcasted_iota(jnp.int32, (MAX_LEN, BLK), 0)
    onehot = (rows == idx[None, :]).astype(jnp.float32)
    proj = lax.dot_general(
        p_ref[...], onehot,
        dimension_numbers=(((0,), (0,)), ((), ())),
        preferred_element_type=jnp.float32,
    )
    out_ref[...] = x_ref[...] + proj


def kernel(x, t, emb_weight, W, b):
    n = x.shape[1]
    num_blk = n // BLK

    p = pl.pallas_call(
        _proj_table_kernel,
        out_shape=jax.ShapeDtypeStruct((MAX_LEN, N_HID), jnp.float32),
    )(emb_weight, W, b.reshape(1, N_HID))

    t3 = t.reshape(num_blk, 1, BLK)

    out = pl.pallas_call(
        _gather_add_kernel,
        grid=(num_blk,),
        in_specs=[
            pl.BlockSpec((1, 1, BLK), lambda i: (i, 0, 0)),
            pl.BlockSpec((N_HID, BLK), lambda i: (0, i)),
            pl.BlockSpec((MAX_LEN, N_HID), lambda i: (0, 0)),
        ],
        out_specs=pl.BlockSpec((N_HID, BLK), lambda i: (0, i)),
        out_shape=jax.ShapeDtypeStruct((N_HID, n), jnp.float32),
        compiler_params=pltpu.CompilerParams(
            dimension_semantics=("arbitrary",),
        ),
    )(t3, x, p)
    return out


# speedup vs baseline: 13.3223x; 1.0334x over previous
import functools

import jax
import jax.numpy as jnp
from jax import lax
from jax.experimental import pallas as pl
from jax.experimental.pallas import tpu as pltpu

N_HID = 128
MAX_LEN = 100
BLK = 12800


def _proj_table_kernel(emb_ref, w_ref, b_ref, p_ref):
    p_ref[...] = lax.dot_general(
        emb_ref[...], w_ref[...],
        dimension_numbers=(((1,), (1,)), ((), ())),
        preferred_element_type=jnp.float32,
    ) + b_ref[...]


def _gather_add_kernel(idx_ref, x_ref, p_ref, out_ref):
    idx = idx_ref[0, 0, :] * 2
    rows = lax.broadcasted_iota(jnp.int32, (MAX_LEN, BLK), 0)
    onehot = (rows == idx[None, :]).astype(jnp.float32)
    proj = lax.dot_general(
        p_ref[...], onehot,
        dimension_numbers=(((0,), (0,)), ((), ())),
        preferred_element_type=jnp.float32,
    )
    out_ref[...] = x_ref[...] + proj


def kernel(x, t, emb_weight, W, b):
    n = x.shape[1]
    num_blk = n // BLK

    p = pl.pallas_call(
        _proj_table_kernel,
        out_shape=jax.ShapeDtypeStruct((MAX_LEN, N_HID), jnp.float32),
    )(emb_weight, W, b.reshape(1, N_HID))

    t3 = t.reshape(num_blk, 1, BLK)

    out = pl.pallas_call(
        _gather_add_kernel,
        grid=(num_blk,),
        in_specs=[
            pl.BlockSpec((1, 1, BLK), lambda i: (i, 0, 0)),
            pl.BlockSpec((N_HID, BLK), lambda i: (0, i)),
            pl.BlockSpec((MAX_LEN, N_HID), lambda i: (0, 0)),
        ],
        out_specs=pl.BlockSpec((N_HID, BLK), lambda i: (0, i)),
        out_shape=jax.ShapeDtypeStruct((N_HID, n), jnp.float32),
        compiler_params=pltpu.CompilerParams(
            dimension_semantics=("arbitrary",),
        ),
    )(t3, x, p)
    return out


# speedup vs baseline: 13.6118x; 1.0217x over previous
import functools

import jax
import jax.numpy as jnp
from jax import lax
from jax.experimental import pallas as pl
from jax.experimental.pallas import tpu as pltpu

N_HID = 128
MAX_LEN = 100
BLK = 20480


def _proj_table_kernel(emb_ref, w_ref, b_ref, p_ref):
    p_ref[...] = lax.dot_general(
        emb_ref[...], w_ref[...],
        dimension_numbers=(((1,), (1,)), ((), ())),
        preferred_element_type=jnp.float32,
    ) + b_ref[...]


def _gather_add_kernel(idx_ref, x_ref, p_ref, out_ref):
    idx = idx_ref[0, 0, :] * 2
    rows = lax.broadcasted_iota(jnp.int32, (MAX_LEN, BLK), 0)
    onehot = (rows == idx[None, :]).astype(jnp.float32)
    proj = lax.dot_general(
        p_ref[...], onehot,
        dimension_numbers=(((0,), (0,)), ((), ())),
        preferred_element_type=jnp.float32,
    )
    out_ref[...] = x_ref[...] + proj


def kernel(x, t, emb_weight, W, b):
    n = x.shape[1]
    num_blk = n // BLK

    p = pl.pallas_call(
        _proj_table_kernel,
        out_shape=jax.ShapeDtypeStruct((MAX_LEN, N_HID), jnp.float32),
    )(emb_weight, W, b.reshape(1, N_HID))

    t3 = t.reshape(num_blk, 1, BLK)

    out = pl.pallas_call(
        _gather_add_kernel,
        grid=(num_blk,),
        in_specs=[
            pl.BlockSpec((1, 1, BLK), lambda i: (i, 0, 0)),
            pl.BlockSpec((N_HID, BLK), lambda i: (0, i)),
            pl.BlockSpec((MAX_LEN, N_HID), lambda i: (0, 0)),
        ],
        out_specs=pl.BlockSpec((N_HID, BLK), lambda i: (0, i)),
        out_shape=jax.ShapeDtypeStruct((N_HID, n), jnp.float32),
        compiler_params=pltpu.CompilerParams(
            dimension_semantics=("arbitrary",),
        ),
    )(t3, x, p)
    return out
